# baseline (speedup 1.0000x reference)
import jax
import jax.numpy as jnp
from jax import lax
from jax.experimental import pallas as pl
from jax.experimental.pallas import tpu as pltpu
from jax.experimental.pallas import tpu_sc as plsc

_B, _T, _D = 4, 16384, 128
_NC, _NS = 2, 16
_NW = _NC * _NS
_TSC = 4096
_TTC = _T - _TSC
_TPW = _TSC // _NW
_C = 64
_NCHUNK = _TPW // _C
_J = _D // 16
_DEPTH = 2
_TBLK = 1024


def _sc_body(enc_hbm, tab_hbm, pos_hbm, out_hbm, idx_v, rows_v, enc_v, gsem, esem, osem):
    wid = lax.axis_index("s") * _NC + lax.axis_index("c")
    t0g = _TTC + wid * _TPW
    t0l = wid * _TPW
    pltpu.sync_copy(pos_hbm.at[pl.ds(t0g, _TPW)], idx_v)

    def start_gather(ci):
        return pltpu.async_copy(
            tab_hbm.at[idx_v.at[pl.ds(ci * _C, _C)]], rows_v.at[ci % _DEPTH], gsem)

    def start_enc_in(ci):
        return pltpu.async_copy(
            enc_hbm.at[pl.ds(0, _B), pl.ds(t0g + ci * _C, _C)],
            enc_v.at[ci % _DEPTH], esem)

    def start_out(ci):
        return pltpu.async_copy(
            enc_v.at[ci % _DEPTH],
            out_hbm.at[pl.ds(0, _B), pl.ds(t0l + ci * _C, _C)], osem)

    g_d = [start_gather(ci) for ci in range(_DEPTH)]
    e_d = [start_enc_in(ci) for ci in range(_DEPTH)]
    o_d = [None] * _NCHUNK
    for ci in range(_NCHUNK):
        cur = ci % _DEPTH
        if ci >= _DEPTH:
            o_d[ci - _DEPTH].wait()
        if ci + _DEPTH < _NCHUNK:
            g_d.append(start_gather(ci + _DEPTH))
            e_d.append(start_enc_in(ci + _DEPTH))
        g_d[ci].wait()
        e_d[ci].wait()

        @plsc.parallel_loop(0, _C)
        def _row(i):
            r = [rows_v[cur, i, pl.ds(j * 16, 16)] for j in range(_J)]
            for b in range(_B):
                for j in range(_J):
                    s = pl.ds(j * 16, 16)
                    plsc.addupdate(enc_v.at[cur, b, i, s], r[j])

        o_d[ci] = start_out(ci)
    for ci in range(max(_NCHUNK - _DEPTH, 0), _NCHUNK):
        o_d[ci].wait()


def _tc_body(enc_ref, tab_ref, out_ref):
    out_ref[...] = enc_ref[...] + tab_ref[...][None, :, :]


def kernel(encoded_tokens, position_table, positions):
    mesh = plsc.VectorSubcoreMesh(
        core_axis_name="c", subcore_axis_name="s",
        num_cores=_NC, num_subcores=_NS,
    )
    sc_run = pl.kernel(
        _sc_body,
        out_type=jax.ShapeDtypeStruct((_B, _TSC, _D), jnp.float32),
        mesh=mesh,
        scratch_types=[
            pltpu.VMEM((_TPW,), jnp.int32),
            pltpu.VMEM((_DEPTH, _C, _D), jnp.float32),
            pltpu.VMEM((_DEPTH, _B, _C, _D), jnp.float32),
            pltpu.SemaphoreType.DMA,
            pltpu.SemaphoreType.DMA,
            pltpu.SemaphoreType.DMA,
        ],
    )
    out_sc = sc_run(encoded_tokens, position_table, positions)

    out_tc = pl.pallas_call(
        _tc_body,
        grid=(_TTC // _TBLK,),
        in_specs=[
            pl.BlockSpec((_B, _TBLK, _D), lambda i: (0, i, 0)),
            pl.BlockSpec((_TBLK, _D), lambda i: (i, 0)),
        ],
        out_specs=pl.BlockSpec((_B, _TBLK, _D), lambda i: (0, i, 0)),
        out_shape=jax.ShapeDtypeStruct((_B, _T, _D), jnp.float32),
    )(encoded_tokens, position_table)

    return lax.dynamic_update_slice(out_tc, out_sc, (0, _TTC, 0))


# speedup vs baseline: 1.0201x; 1.0201x over previous
import jax
import jax.numpy as jnp
from jax import lax
from jax.experimental import pallas as pl
from jax.experimental.pallas import tpu as pltpu
from jax.experimental.pallas import tpu_sc as plsc

_B, _T, _D = 4, 16384, 128
_NC, _NS = 2, 16
_NW = _NC * _NS
_TPW = _T // _NW
_C = 64
_NCHUNK = _TPW // _C
_J = _D // 16
_DEPTH = 3


def _sc_body(enc_hbm, tab_hbm, pos_hbm, out_hbm, idx_v, rows_v, enc_v, gsem, esem, osem):
    wid = lax.axis_index("s") * _NC + lax.axis_index("c")
    t0 = wid * _TPW
    pltpu.sync_copy(pos_hbm.at[pl.ds(t0, _TPW)], idx_v)

    def start_gather(ci):
        return pltpu.async_copy(
            tab_hbm.at[idx_v.at[pl.ds(ci * _C, _C)]], rows_v.at[ci % _DEPTH], gsem)

    def start_enc_in(ci):
        tc0 = t0 + ci * _C
        return [pltpu.async_copy(enc_hbm.at[pl.ds(0, _B), pl.ds(tc0, _C)],
                                 enc_v.at[ci % _DEPTH], esem)]

    def start_out(ci):
        tc0 = t0 + ci * _C
        return [pltpu.async_copy(enc_v.at[ci % _DEPTH],
                                 out_hbm.at[pl.ds(0, _B), pl.ds(tc0, _C)], osem)]

    g_d = [start_gather(ci) for ci in range(2)]
    e_d = [start_enc_in(ci) for ci in range(2)]
    o_d = [None] * _NCHUNK
    for ci in range(_NCHUNK):
        cur = ci % _DEPTH
        if ci >= 2:
            for d in o_d[ci - 2]:
                d.wait()
        if ci + 1 < _NCHUNK and ci >= 1:
            g_d.append(start_gather(ci + 1))
            e_d.append(start_enc_in(ci + 1))
        g_d[ci].wait()
        for d in e_d[ci]:
            d.wait()

        @plsc.parallel_loop(0, _C)
        def _row(i):
            r = [rows_v[cur, i, pl.ds(j * 16, 16)] for j in range(_J)]
            for b in range(_B):
                for j in range(_J):
                    s = pl.ds(j * 16, 16)
                    plsc.addupdate(enc_v.at[cur, b, i, s], r[j])

        o_d[ci] = start_out(ci)
    for ci in (_NCHUNK - 2, _NCHUNK - 1):
        for d in o_d[ci]:
            d.wait()


def kernel(encoded_tokens, position_table, positions):
    mesh = plsc.VectorSubcoreMesh(
        core_axis_name="c", subcore_axis_name="s",
        num_cores=_NC, num_subcores=_NS,
    )
    run = pl.kernel(
        _sc_body,
        out_type=jax.ShapeDtypeStruct((_B, _T, _D), jnp.float32),
        mesh=mesh,
        scratch_types=[
            pltpu.VMEM((_TPW,), jnp.int32),
            pltpu.VMEM((_DEPTH, _C, _D), jnp.float32),
            pltpu.VMEM((_DEPTH, _B, _C, _D), jnp.float32),
            pltpu.SemaphoreType.DMA,
            pltpu.SemaphoreType.DMA,
            pltpu.SemaphoreType.DMA,
        ],
    )
    return run(encoded_tokens, position_table, positions)
